# baseline (speedup 1.0000x reference)
import jax
import jax.numpy as jnp
from jax import lax
from jax.experimental import pallas as pl

K = 512
T_BLK = 512
ROT = ((13, 15, 26, 6), (17, 29, 16, 24))


def _threefry_bits(flat):
    k1 = jnp.uint32(0)
    k2 = jnp.uint32(42)
    ks = (k1, k2, k1 ^ k2 ^ jnp.uint32(0x1BD11BDA))
    x0 = jnp.zeros_like(flat) + ks[0]
    x1 = flat + ks[1]
    for i in range(5):
        for r in ROT[i % 2]:
            x0 = x0 + x1
            x1 = ((x1 << jnp.uint32(r)) | (x1 >> jnp.uint32(32 - r))) ^ x0
        x0 = x0 + ks[(i + 1) % 3]
        x1 = x1 + ks[(i + 2) % 3] + jnp.uint32(i + 1)
    return x0 ^ x1


def _gumbel(sh):
    tiny = jnp.float32(1.1754943508222875e-38)
    u = jnp.where(sh == 0, tiny, sh.astype(jnp.float32) * jnp.float32(2.0 ** -23))
    return -jnp.log(-jnp.log(u))


def _sample_kernel(x_ref, qrow_ref, out_ref):
    g = pl.program_id(0)
    k_iota = lax.broadcasted_iota(jnp.uint32, (K, T_BLK), 0)
    t_iota = lax.broadcasted_iota(jnp.uint32, (K, T_BLK), 1)
    base = jnp.uint32(g) * jnp.uint32(T_BLK)
    flat = (base + t_iota) * jnp.uint32(K) + k_iota
    bits = _threefry_bits(flat)
    shifted = (bits >> jnp.uint32(9)).astype(jnp.int32)

    xs = x_ref[0]
    kk = k_iota.astype(jnp.int32)
    is_x = kk == xs
    masked = jnp.where(is_x, -1, shifted)
    maxv = jnp.max(masked, axis=0, keepdims=True)
    m_idx = jnp.min(jnp.where(masked == maxv, kk, K), axis=0, keepdims=True)
    s_x = jnp.max(jnp.where(is_x, shifted, -1), axis=0, keepdims=True)

    log_diag = jnp.log(jnp.maximum(qrow_ref[0, 0, 0], jnp.float32(1e-12)))
    log_off = jnp.log(jnp.maximum(qrow_ref[0, 0, 1], jnp.float32(1e-12)))
    a_other = _gumbel(maxv) + log_off
    a_self = _gumbel(s_x) + log_diag
    take_x = (a_self > a_other) | ((a_self == a_other) & (xs < m_idx))
    out_ref[0] = jnp.where(take_x, xs, m_idx)


def kernel(x, t, qtcum):
    qrow = lax.dynamic_slice(qtcum, (t, 0, 0), (1, 1, 128))
    x3 = x.astype(jnp.int32).reshape(128, 1, 512)
    out = pl.pallas_call(
        _sample_kernel,
        grid=(128,),
        in_specs=[
            pl.BlockSpec((1, 1, 512), lambda g: (g, 0, 0)),
            pl.BlockSpec((1, 1, 128), lambda g: (0, 0, 0)),
        ],
        out_specs=pl.BlockSpec((1, 1, 512), lambda g: (g, 0, 0)),
        out_shape=jax.ShapeDtypeStruct((128, 1, 512), jnp.int32),
    )(x3, qrow)
    return out.reshape(128, 512)


# speedup vs baseline: 1.7085x; 1.7085x over previous
import jax
import jax.numpy as jnp
from jax import lax
from jax.experimental import pallas as pl

K = 512
T_BLK = 512
ROT = ((13, 15, 26, 6), (17, 29, 16, 24))


def _threefry_bits(flat):
    k1 = jnp.uint32(0)
    k2 = jnp.uint32(42)
    ks = (k1, k2, k1 ^ k2 ^ jnp.uint32(0x1BD11BDA))
    x1 = flat + ks[1]
    x0 = x1
    x1 = ((x1 << jnp.uint32(13)) | (x1 >> jnp.uint32(19))) ^ x0
    first = True
    for i in range(5):
        for r in ROT[i % 2]:
            if first:
                first = False
                continue
            x0 = x0 + x1
            x1 = ((x1 << jnp.uint32(r)) | (x1 >> jnp.uint32(32 - r))) ^ x0
        x0 = x0 + ks[(i + 1) % 3]
        x1 = x1 + ks[(i + 2) % 3] + jnp.uint32(i + 1)
    return x0 ^ x1


def _gumbel(sh):
    tiny = jnp.float32(1.1754943508222875e-38)
    u = jnp.where(sh == 0, tiny, sh.astype(jnp.float32) * jnp.float32(2.0 ** -23))
    return -jnp.log(-jnp.log(u))


CHUNK = 8


def _sample_kernel(x_ref, qrow_ref, out_ref):
    g = pl.program_id(0)
    xs = x_ref[0]
    t_iota = lax.broadcasted_iota(jnp.uint32, (CHUNK, T_BLK), 1)
    p_iota = lax.broadcasted_iota(jnp.uint32, (CHUNK, T_BLK), 0)
    f0 = t_iota * jnp.uint32(K) + p_iota
    kk0 = p_iota.astype(jnp.int32)
    base = jnp.uint32(g) * jnp.uint32(T_BLK * K)

    run_max = jnp.full((CHUNK, T_BLK), -1, jnp.int32)
    run_idx = jnp.zeros((CHUNK, T_BLK), jnp.int32)
    run_sx = jnp.full((CHUNK, T_BLK), -1, jnp.int32)
    for c in range(K // CHUNK):
        flat = f0 + (base + jnp.uint32(c * CHUNK))
        bits = _threefry_bits(flat)
        shifted = (bits >> jnp.uint32(9)).astype(jnp.int32)
        kk = kk0 + jnp.int32(c * CHUNK)
        is_x = kk == xs
        masked = jnp.where(is_x, -1, shifted)
        run_sx = jnp.maximum(run_sx, jnp.where(is_x, shifted, -1))
        take = masked > run_max
        run_max = jnp.maximum(run_max, masked)
        run_idx = jnp.where(take, kk, run_idx)

    maxv = jnp.max(run_max, axis=0, keepdims=True)
    m_idx = jnp.min(jnp.where(run_max == maxv, run_idx, K), axis=0,
                    keepdims=True)
    s_x = jnp.max(run_sx, axis=0, keepdims=True)

    log_diag = jnp.log(jnp.maximum(qrow_ref[0, 0, 0], jnp.float32(1e-12)))
    log_off = jnp.log(jnp.maximum(qrow_ref[0, 0, 1], jnp.float32(1e-12)))
    a_other = _gumbel(maxv) + log_off
    a_self = _gumbel(s_x) + log_diag
    take_x = (a_self > a_other) | ((a_self == a_other) & (xs < m_idx))
    out_ref[0] = jnp.where(take_x, xs, m_idx)


def kernel(x, t, qtcum):
    qrow = lax.dynamic_slice(qtcum, (t, 0, 0), (1, 1, 128))
    x3 = x.astype(jnp.int32).reshape(128, 1, 512)
    out = pl.pallas_call(
        _sample_kernel,
        grid=(128,),
        in_specs=[
            pl.BlockSpec((1, 1, 512), lambda g: (g, 0, 0)),
            pl.BlockSpec((1, 1, 128), lambda g: (0, 0, 0)),
        ],
        out_specs=pl.BlockSpec((1, 1, 512), lambda g: (g, 0, 0)),
        out_shape=jax.ShapeDtypeStruct((128, 1, 512), jnp.int32),
    )(x3, qrow)
    return out.reshape(128, 512)


# speedup vs baseline: 1.7648x; 1.0330x over previous
import jax
import jax.numpy as jnp
from jax import lax
from jax.experimental import pallas as pl

K = 512
T_BLK = 1024
ROT = ((13, 15, 26, 6), (17, 29, 16, 24))


def _threefry_bits(flat):
    k1 = jnp.uint32(0)
    k2 = jnp.uint32(42)
    ks = (k1, k2, k1 ^ k2 ^ jnp.uint32(0x1BD11BDA))
    x1 = flat + ks[1]
    x0 = x1
    x1 = ((x1 << jnp.uint32(13)) | (x1 >> jnp.uint32(19))) ^ x0
    first = True
    for i in range(5):
        for r in ROT[i % 2]:
            if first:
                first = False
                continue
            x0 = x0 + x1
            x1 = ((x1 << jnp.uint32(r)) | (x1 >> jnp.uint32(32 - r))) ^ x0
        x0 = x0 + ks[(i + 1) % 3]
        x1 = x1 + ks[(i + 2) % 3] + jnp.uint32(i + 1)
    return x0 ^ x1


def _gumbel(sh):
    tiny = jnp.float32(1.1754943508222875e-38)
    u = jnp.where(sh == 0, tiny, sh.astype(jnp.float32) * jnp.float32(2.0 ** -23))
    return -jnp.log(-jnp.log(u))


CHUNK = 8


def _sample_kernel(x_ref, qrow_ref, out_ref):
    g = pl.program_id(0)
    xs = x_ref[0]
    t_iota = lax.broadcasted_iota(jnp.uint32, (CHUNK, T_BLK), 1)
    p_iota = lax.broadcasted_iota(jnp.uint32, (CHUNK, T_BLK), 0)
    f0 = t_iota * jnp.uint32(K) + p_iota
    inv0 = jnp.uint32(K - 1) - p_iota
    base = jnp.uint32(g) * jnp.uint32(T_BLK * K)

    run = jnp.full((CHUNK, T_BLK), -(2 ** 31), jnp.int32)
    for c in range(K // CHUNK):
        flat = f0 + (base + jnp.uint32(c * CHUNK))
        bits = _threefry_bits(flat)
        packed = (((bits ^ jnp.uint32(0x80000000)) & jnp.uint32(0xFFFFFE00))
                  | (inv0 - jnp.uint32(c * CHUNK)))
        run = jnp.maximum(run, packed.astype(jnp.int32))

    comb_s = jnp.max(run, axis=0, keepdims=True)
    comb = comb_s.astype(jnp.uint32) ^ jnp.uint32(0x80000000)
    m_idx = (jnp.int32(K - 1) - (comb & jnp.uint32(0x1FF)).astype(jnp.int32))
    s_m = (comb >> jnp.uint32(9)).astype(jnp.int32)

    flat_x = (base + t_iota[0:1] * jnp.uint32(K)) + xs.astype(jnp.uint32)
    s_x = (_threefry_bits(flat_x) >> jnp.uint32(9)).astype(jnp.int32)

    log_diag = jnp.log(jnp.maximum(qrow_ref[0, 0, 0], jnp.float32(1e-12)))
    log_off = jnp.log(jnp.maximum(qrow_ref[0, 0, 1], jnp.float32(1e-12)))
    a_other = _gumbel(s_m) + log_off
    a_self = _gumbel(s_x) + log_diag
    take_x = (a_self > a_other) | ((a_self == a_other) & (xs < m_idx))
    out_ref[0] = jnp.where(take_x, xs, m_idx)


def kernel(x, t, qtcum):
    qrow = lax.dynamic_slice(qtcum, (t, 0, 0), (1, 1, 128))
    x3 = x.astype(jnp.int32).reshape(64, 1, 1024)
    out = pl.pallas_call(
        _sample_kernel,
        grid=(64,),
        in_specs=[
            pl.BlockSpec((1, 1, 1024), lambda g: (g, 0, 0)),
            pl.BlockSpec((1, 1, 128), lambda g: (0, 0, 0)),
        ],
        out_specs=pl.BlockSpec((1, 1, 1024), lambda g: (g, 0, 0)),
        out_shape=jax.ShapeDtypeStruct((64, 1, 1024), jnp.int32),
    )(x3, qrow)
    return out.reshape(128, 512)


# speedup vs baseline: 1.8364x; 1.0406x over previous
import jax
import jax.numpy as jnp
from jax import lax
from jax.experimental import pallas as pl

K = 512
T_BLK = 1024
ROT = ((13, 15, 26, 6), (17, 29, 16, 24))


def _threefry_bits(x1, final_bias=0):
    k1 = jnp.uint32(0)
    k2 = jnp.uint32(42)
    ks = (k1, k2, k1 ^ k2 ^ jnp.uint32(0x1BD11BDA))
    assert final_bias in (0, 0x80000000)
    x0 = x1
    x1 = ((x1 << jnp.uint32(13)) | (x1 >> jnp.uint32(19))) ^ x0
    first = True
    for i in range(5):
        for r in ROT[i % 2]:
            if first:
                first = False
                continue
            x0 = x0 + x1
            x1 = ((x1 << jnp.uint32(r)) | (x1 >> jnp.uint32(32 - r))) ^ x0
        x0 = x0 + ks[(i + 1) % 3]
        extra = jnp.uint32(final_bias) if i == 4 else jnp.uint32(0)
        x1 = x1 + (ks[(i + 2) % 3] + jnp.uint32(i + 1) + extra)
    return x0 ^ x1


def _gumbel(sh):
    tiny = jnp.float32(1.1754943508222875e-38)
    u = jnp.where(sh == 0, tiny, sh.astype(jnp.float32) * jnp.float32(2.0 ** -23))
    return -jnp.log(-jnp.log(u))


CHUNK = 8


def _sample_kernel(x_ref, qrow_ref, out_ref):
    g = pl.program_id(0)
    xs = x_ref[0]
    t_iota = lax.broadcasted_iota(jnp.uint32, (CHUNK, T_BLK), 1)
    p_iota = lax.broadcasted_iota(jnp.uint32, (CHUNK, T_BLK), 0)
    f0 = t_iota * jnp.uint32(K) + p_iota
    inv0 = jnp.uint32(K - 1) - p_iota
    base = jnp.uint32(g) * jnp.uint32(T_BLK * K)

    run = jnp.full((CHUNK, T_BLK), -(2 ** 31), jnp.int32)
    for c in range(K // CHUNK):
        x1 = f0 + (base + jnp.uint32(c * CHUNK + 42))
        bits_b = _threefry_bits(x1, final_bias=0x80000000)
        packed = ((bits_b & jnp.uint32(0xFFFFFE00))
                  | (inv0 - jnp.uint32(c * CHUNK)))
        run = jnp.maximum(run, packed.astype(jnp.int32))

    comb_s = jnp.max(run, axis=0, keepdims=True)
    comb = comb_s.astype(jnp.uint32) ^ jnp.uint32(0x80000000)
    m_idx = (jnp.int32(K - 1) - (comb & jnp.uint32(0x1FF)).astype(jnp.int32))
    s_m = (comb >> jnp.uint32(9)).astype(jnp.int32)

    x1x = (f0[0:1] + (base + jnp.uint32(42))) + xs.astype(jnp.uint32)
    s_x = (_threefry_bits(x1x) >> jnp.uint32(9)).astype(jnp.int32)

    log_diag = jnp.log(jnp.maximum(qrow_ref[0, 0, 0], jnp.float32(1e-12)))
    log_off = jnp.log(jnp.maximum(qrow_ref[0, 0, 1], jnp.float32(1e-12)))
    a_other = _gumbel(s_m) + log_off
    a_self = _gumbel(s_x) + log_diag
    take_x = (a_self > a_other) | ((a_self == a_other) & (xs < m_idx))
    out_ref[0] = jnp.where(take_x, xs, m_idx)


def kernel(x, t, qtcum):
    qrow = lax.dynamic_slice(qtcum, (t, 0, 0), (1, 1, 128))
    x3 = x.astype(jnp.int32).reshape(64, 1, 1024)
    out = pl.pallas_call(
        _sample_kernel,
        grid=(64,),
        in_specs=[
            pl.BlockSpec((1, 1, 1024), lambda g: (g, 0, 0)),
            pl.BlockSpec((1, 1, 128), lambda g: (0, 0, 0)),
        ],
        out_specs=pl.BlockSpec((1, 1, 1024), lambda g: (g, 0, 0)),
        out_shape=jax.ShapeDtypeStruct((64, 1, 1024), jnp.int32),
    )(x3, qrow)
    return out.reshape(128, 512)
